# baseline (speedup 1.0000x reference)
import functools

import jax
import jax.numpy as jnp
from jax import lax
from jax.experimental import pallas as pl
from jax.experimental.pallas import tpu as pltpu
from jax.experimental.pallas import tpu_sc as plsc

V = 1000000
D = 64
B = 16384
NW = 32
W = 256
STRIPE = 31232
NBLK = STRIPE // W
MAIN = NW * STRIPE
TAIL0 = 999936
CCAP = 2048
RING = 32
BPW = B // NW

_mesh = plsc.VectorSubcoreMesh(core_axis_name="c", subcore_axis_name="s")


@functools.partial(
    pl.kernel,
    mesh=_mesh,
    compiler_params=pltpu.CompilerParams(needs_layout_passes=False),
    out_type=(
        jax.ShapeDtypeStruct((B * D,), jnp.float32),
        jax.ShapeDtypeStruct((B * D,), jnp.float32),
    ),
    scratch_types=[
        pltpu.VMEM((B,), jnp.int32),
        pltpu.VMEM((B,), jnp.int32),
        pltpu.VMEM((2, D, W), jnp.float32),
        pltpu.VMEM((2, D, W), jnp.float32),
        pltpu.VMEM((CCAP,), jnp.int32),
        pltpu.VMEM((CCAP,), jnp.int32),
        pltpu.VMEM((CCAP,), jnp.int32),
        pltpu.VMEM((CCAP,), jnp.int32),
        pltpu.VMEM((2048,), jnp.int32),
        pltpu.VMEM((2048,), jnp.int32),
        pltpu.VMEM((2048,), jnp.int32),
        pltpu.VMEM((2048,), jnp.int32),
        pltpu.VMEM((32,), jnp.int32),
        pltpu.VMEM((32,), jnp.int32),
        pltpu.VMEM((RING, D), jnp.float32),
        pltpu.VMEM((4096,), jnp.float32),
        pltpu.VMEM((4096,), jnp.float32),
        pltpu.SemaphoreType.DMA,
        pltpu.SemaphoreType.DMA,
        pltpu.SemaphoreType.DMA,
    ],
)
def _stream_compact(users_hbm, items_hbm, ut_hbm, it_hbm,
                    utail_hbm, itail_hbm, uout_hbm, iout_hbm,
                    uidx_v, iidx_v, ublk_v, iblk_v,
                    ucidx_v, ucpos_v, icidx_v, icpos_v,
                    usidx_v, uspos_v, isidx_v, ispos_v,
                    hidx_v, hpos_v, ring_v,
                    utail_v, itail_v, usem, isem, osem):
    wid = lax.axis_index("s") * 2 + lax.axis_index("c")
    lo = wid * STRIPE
    is_last = (wid == NW - 1).astype(jnp.int32)
    hi = lo + STRIPE + is_last * (2 * W)
    nblk = NBLK + is_last * 2

    def fire(j, buf):
        off = lo + j * W
        pltpu.async_copy(ut_hbm.at[:, pl.ds(off, W)], ublk_v.at[buf], usem)
        pltpu.async_copy(it_hbm.at[:, pl.ds(off, W)], iblk_v.at[buf], isem)

    fire(0, 0)
    fire(1, 1)

    pltpu.sync_copy(users_hbm, uidx_v)
    pltpu.sync_copy(items_hbm, iidx_v)
    pltpu.sync_copy(utail_hbm, utail_v)
    pltpu.sync_copy(itail_hbm, itail_v)

    lane = lax.iota(jnp.int32, 16)

    def popcnt(m):
        return plsc.all_reduce_population_count(m)[0]

    def l1(idx_ref, cidx_ref, cpos_ref):
        def body(v, off):
            idx = idx_ref[pl.ds(v * 16, 16)]
            m = (idx >= lo) & (idx < hi)
            cnt = popcnt(m)
            o = jnp.minimum(off, CCAP - 16)
            plsc.store_compressed(cidx_ref.at[pl.ds(o, 16)], idx, mask=m)
            plsc.store_compressed(cpos_ref.at[pl.ds(o, 16)], v * 16 + lane, mask=m)
            return off + cnt
        return lax.fori_loop(0, B // 16, body, jnp.int32(0))

    ucnt = l1(uidx_v, ucidx_v, ucpos_v)
    icnt = l1(iidx_v, icidx_v, icpos_v)

    SUBN, SUBCAP, SUBW = 8, 256, 16 * W

    def partition(cidx_ref, cpos_ref, cnt, sidx_ref, spos_ref):
        ncv = lax.div(cnt + 15, jnp.int32(16))
        cnts = []
        for s in range(SUBN):
            s_lo = lo + s * SUBW
            s_hi = jnp.minimum(lo + (s + 1) * SUBW, hi)

            def body(cv, off, s_lo=s_lo, s_hi=s_hi, s=s):
                gid = cv * 16 + lane
                ci = cidx_ref[pl.ds(cv * 16, 16)]
                cp = cpos_ref[pl.ds(cv * 16, 16)]
                m = (gid < cnt) & (ci >= s_lo) & (ci < s_hi)
                c = popcnt(m)
                o = jnp.minimum(off, SUBCAP - 16)
                plsc.store_compressed(
                    sidx_ref.at[pl.ds(s * SUBCAP + o, 16)], ci, mask=m)
                plsc.store_compressed(
                    spos_ref.at[pl.ds(s * SUBCAP + o, 16)], cp, mask=m)
                return off + c
            cnts.append(lax.fori_loop(0, ncv, body, jnp.int32(0)))
        cntvec = jnp.zeros_like(lane)
        for s in range(SUBN):
            cntvec = cntvec + jnp.where(lane == s, cnts[s], 0)
        return cntvec

    ucntv = partition(ucidx_v, ucpos_v, ucnt, usidx_v, uspos_v)
    icntv = partition(icidx_v, icpos_v, icnt, isidx_v, ispos_v)

    def scan_block(blk_ref, j, boff, sidx_ref, spos_ref, cntvec,
                   out_hbm, issued):
        s_dyn = lax.shift_right_logical(j, 4)
        sbase = s_dyn * SUBCAP
        cnt = jnp.max(jnp.where(lane == s_dyn, cntvec, 0))
        ncv = lax.div(cnt + 15, jnp.int32(16))

        def cand_body(cv, issued):
            gid = cv * 16 + lane
            cidx = sidx_ref[pl.ds(sbase + cv * 16, 16)]
            cpos = spos_ref[pl.ds(sbase + cv * 16, 16)]
            m = (gid < cnt) & (cidx >= boff) & (cidx < boff + W)
            nh = popcnt(m)
            plsc.store_compressed(hidx_v.at[pl.ds(0, 16)], cidx - boff, mask=m)
            plsc.store_compressed(hpos_v.at[pl.ds(0, 16)], cpos, mask=m)

            def hit_body(h, issued):
                col_s = hidx_v[pl.ds(h, 16)][0]
                pos_s = hpos_v[pl.ds(h, 16)][0]
                col_v = jnp.zeros_like(lane) + col_s
                slot = lax.rem(issued, RING)
                @pl.when(issued >= RING)
                def _():
                    pltpu.make_async_copy(
                        out_hbm.at[pl.ds(0, D)], ring_v.at[0], osem).wait()
                for g in range(D // 16):
                    val = plsc.load_gather(blk_ref, [g * 16 + lane, col_v])
                    ring_v[slot, pl.ds(g * 16, 16)] = val
                pltpu.async_copy(
                    ring_v.at[slot], out_hbm.at[pl.ds(pos_s * D, D)], osem)
                return issued + 1

            return lax.fori_loop(0, nh, hit_body, issued)

        return lax.fori_loop(0, ncv, cand_body, issued)

    def blk_body(j, issued):
        buf = lax.rem(j, 2)
        boff = lo + j * W
        pltpu.make_async_copy(
            ut_hbm.at[:, pl.ds(0, W)], ublk_v.at[buf], usem).wait()
        pltpu.make_async_copy(
            it_hbm.at[:, pl.ds(0, W)], iblk_v.at[buf], isem).wait()
        issued = scan_block(ublk_v.at[buf], j, boff, usidx_v, uspos_v,
                            ucntv, uout_hbm, issued)
        issued = scan_block(iblk_v.at[buf], j, boff, isidx_v, ispos_v,
                            icntv, iout_hbm, issued)
        @pl.when(j + 2 < nblk)
        def _():
            fire(j + 2, buf)
        return issued

    issued = lax.fori_loop(0, nblk, blk_body, jnp.int32(0))

    def tail_scan(idx_ref, tail_ref, out_hbm, issued):
        def body(v, issued):
            base = wid * BPW + v * 16
            idx = idx_ref[pl.ds(base, 16)]
            m = idx >= TAIL0
            nh = popcnt(m)
            plsc.store_compressed(hidx_v.at[pl.ds(0, 16)],
                                  (idx - TAIL0) * D, mask=m)
            plsc.store_compressed(hpos_v.at[pl.ds(0, 16)], base + lane, mask=m)

            def hit_body(h, issued):
                adr_s = hidx_v[pl.ds(h, 16)][0]
                pos_s = hpos_v[pl.ds(h, 16)][0]
                slot = lax.rem(issued, RING)
                @pl.when(issued >= RING)
                def _():
                    pltpu.make_async_copy(
                        out_hbm.at[pl.ds(0, D)], ring_v.at[0], osem).wait()
                for g in range(D // 16):
                    val = plsc.load_gather(tail_ref,
                                           [adr_s + g * 16 + lane])
                    ring_v[slot, pl.ds(g * 16, 16)] = val
                pltpu.async_copy(
                    ring_v.at[slot], out_hbm.at[pl.ds(pos_s * D, D)], osem)
                return issued + 1

            return lax.fori_loop(0, nh, hit_body, issued)
        return lax.fori_loop(0, BPW // 16, body, issued)

    issued = tail_scan(uidx_v, utail_v, uout_hbm, issued)
    issued = tail_scan(iidx_v, itail_v, iout_hbm, issued)

    def drain(_, carry):
        pltpu.make_async_copy(
            uout_hbm.at[pl.ds(0, D)], ring_v.at[0], osem).wait()
        return carry
    lax.fori_loop(0, jnp.minimum(issued, RING), drain, jnp.int32(0))


@functools.partial(
    pl.kernel,
    mesh=_mesh,
    compiler_params=pltpu.CompilerParams(needs_layout_passes=False),
    out_type=(
        jax.ShapeDtypeStruct((D, B), jnp.float32),
        jax.ShapeDtypeStruct((D, B), jnp.float32),
    ),
    scratch_types=[
        pltpu.VMEM((BPW * D,), jnp.float32),
        pltpu.VMEM((D, BPW), jnp.float32),
        pltpu.SemaphoreType.DMA,
    ],
)
def _retile(u1d_hbm, i1d_hbm, uout_hbm, iout_hbm, flat_v, outT_v, sem):
    wid = lax.axis_index("s") * 2 + lax.axis_index("c")
    lane = lax.iota(jnp.int32, 16)

    def one(src_hbm, dst_hbm):
        pltpu.sync_copy(src_hbm.at[pl.ds(wid * BPW * D, BPW * D)], flat_v)

        def body(g, carry):
            for c in range(D):
                val = plsc.load_gather(
                    flat_v, [(g * 16 + lane) * D + c])
                outT_v[c, pl.ds(g * 16, 16)] = val
            return carry
        lax.fori_loop(0, BPW // 16, body, 0)
        pltpu.sync_copy(outT_v, dst_hbm.at[:, pl.ds(wid * BPW, BPW)])

    one(u1d_hbm, uout_hbm)
    one(i1d_hbm, iout_hbm)


@jax.jit
def kernel(users, items, user_table, item_table):
    utail = user_table[TAIL0:].reshape(-1)
    itail = item_table[TAIL0:].reshape(-1)
    u1d, i1d = _stream_compact(users, items, user_table.T, item_table.T,
                               utail, itail)
    uoT, ioT = _retile(u1d, i1d)
    return (uoT.T, ioT.T)


# speedup vs baseline: 1.1229x; 1.1229x over previous
import functools

import jax
import jax.numpy as jnp
from jax import lax
from jax.experimental import pallas as pl
from jax.experimental.pallas import tpu as pltpu
from jax.experimental.pallas import tpu_sc as plsc

V = 1000000
D = 64
B = 16384
NW = 32
W = 256
STRIPE = 31232
NBLK = STRIPE // W
MAIN = NW * STRIPE
TAIL0 = 999936
CCAP = 2048
RING = 32
BPW = B // NW

_mesh = plsc.VectorSubcoreMesh(core_axis_name="c", subcore_axis_name="s")


@functools.partial(
    pl.kernel,
    mesh=_mesh,
    compiler_params=pltpu.CompilerParams(needs_layout_passes=False),
    out_type=(
        jax.ShapeDtypeStruct((B * D,), jnp.float32),
        jax.ShapeDtypeStruct((B * D,), jnp.float32),
    ),
    scratch_types=[
        pltpu.VMEM((B,), jnp.int32),
        pltpu.VMEM((B,), jnp.int32),
        pltpu.VMEM((2, D, W), jnp.float32),
        pltpu.VMEM((2, D, W), jnp.float32),
        pltpu.VMEM((CCAP,), jnp.int32),
        pltpu.VMEM((CCAP,), jnp.int32),
        pltpu.VMEM((CCAP,), jnp.int32),
        pltpu.VMEM((CCAP,), jnp.int32),
        pltpu.VMEM((2048,), jnp.int32),
        pltpu.VMEM((2048,), jnp.int32),
        pltpu.VMEM((2048,), jnp.int32),
        pltpu.VMEM((2048,), jnp.int32),
        pltpu.VMEM((32,), jnp.int32),
        pltpu.VMEM((32,), jnp.int32),
        pltpu.VMEM((RING, D), jnp.float32),
        pltpu.VMEM((4096,), jnp.float32),
        pltpu.VMEM((4096,), jnp.float32),
        pltpu.SemaphoreType.DMA,
        pltpu.SemaphoreType.DMA,
        pltpu.SemaphoreType.DMA,
    ],
)
def _stream_compact(users_hbm, items_hbm, ut_hbm, it_hbm,
                    utail_hbm, itail_hbm, uout_hbm, iout_hbm,
                    uidx_v, iidx_v, ublk_v, iblk_v,
                    ucidx_v, ucpos_v, icidx_v, icpos_v,
                    usidx_v, uspos_v, isidx_v, ispos_v,
                    hidx_v, hpos_v, ring_v,
                    utail_v, itail_v, usem, isem, osem):
    wid = lax.axis_index("s") * 2 + lax.axis_index("c")
    lo = wid * STRIPE
    is_last = (wid == NW - 1).astype(jnp.int32)
    hi = lo + STRIPE + is_last * (2 * W)
    nblk = NBLK + is_last * 2

    def fire(j, buf):
        off = lo + j * W
        pltpu.async_copy(ut_hbm.at[:, pl.ds(off, W)], ublk_v.at[buf], usem)
        pltpu.async_copy(it_hbm.at[:, pl.ds(off, W)], iblk_v.at[buf], isem)

    fire(0, 0)
    fire(1, 1)

    pltpu.sync_copy(users_hbm, uidx_v)
    pltpu.sync_copy(items_hbm, iidx_v)
    pltpu.sync_copy(utail_hbm, utail_v)
    pltpu.sync_copy(itail_hbm, itail_v)

    lane = lax.iota(jnp.int32, 16)

    def popcnt(m):
        return plsc.all_reduce_population_count(m)[0]

    def l1(idx_ref, cidx_ref, cpos_ref):
        def body(v, off):
            idx = idx_ref[pl.ds(v * 16, 16)]
            m = (idx >= lo) & (idx < hi)
            cnt = popcnt(m)
            o = jnp.minimum(off, CCAP - 16)
            plsc.store_compressed(cidx_ref.at[pl.ds(o, 16)], idx, mask=m)
            plsc.store_compressed(cpos_ref.at[pl.ds(o, 16)], v * 16 + lane, mask=m)
            return off + cnt
        return lax.fori_loop(0, B // 16, body, jnp.int32(0))

    ucnt = l1(uidx_v, ucidx_v, ucpos_v)
    icnt = l1(iidx_v, icidx_v, icpos_v)

    SUBN, SUBCAP, SUBW = 8, 256, 16 * W

    def partition(cidx_ref, cpos_ref, cnt, sidx_ref, spos_ref):
        ncv = lax.div(cnt + 15, jnp.int32(16))
        cnts = []
        for s in range(SUBN):
            s_lo = lo + s * SUBW
            s_hi = jnp.minimum(lo + (s + 1) * SUBW, hi)

            def body(cv, off, s_lo=s_lo, s_hi=s_hi, s=s):
                gid = cv * 16 + lane
                ci = cidx_ref[pl.ds(cv * 16, 16)]
                cp = cpos_ref[pl.ds(cv * 16, 16)]
                m = (gid < cnt) & (ci >= s_lo) & (ci < s_hi)
                c = popcnt(m)
                o = jnp.minimum(off, SUBCAP - 16)
                plsc.store_compressed(
                    sidx_ref.at[pl.ds(s * SUBCAP + o, 16)], ci, mask=m)
                plsc.store_compressed(
                    spos_ref.at[pl.ds(s * SUBCAP + o, 16)], cp, mask=m)
                return off + c
            cnts.append(lax.fori_loop(0, ncv, body, jnp.int32(0)))
        cntvec = jnp.zeros_like(lane)
        for s in range(SUBN):
            cntvec = cntvec + jnp.where(lane == s, cnts[s], 0)
        return cntvec

    ucntv = partition(ucidx_v, ucpos_v, ucnt, usidx_v, uspos_v)
    icntv = partition(icidx_v, icpos_v, icnt, isidx_v, ispos_v)

    def scan_block(blk_ref, j, boff, sidx_ref, spos_ref, cntvec,
                   out_hbm, issued):
        s_dyn = lax.shift_right_logical(j, 4)
        sbase = s_dyn * SUBCAP
        cnt = jnp.max(jnp.where(lane == s_dyn, cntvec, 0))
        ncv = lax.div(cnt + 15, jnp.int32(16))

        def cand_body(cv, issued):
            gid = cv * 16 + lane
            cidx = sidx_ref[pl.ds(sbase + cv * 16, 16)]
            cpos = spos_ref[pl.ds(sbase + cv * 16, 16)]
            m = (gid < cnt) & (cidx >= boff) & (cidx < boff + W)
            nh = popcnt(m)
            plsc.store_compressed(hidx_v.at[pl.ds(0, 16)], cidx - boff, mask=m)
            plsc.store_compressed(hpos_v.at[pl.ds(0, 16)], cpos, mask=m)

            def hit_body(h, issued):
                col_s = hidx_v[pl.ds(h, 16)][0]
                pos_s = hpos_v[pl.ds(h, 16)][0]
                col_v = jnp.zeros_like(lane) + col_s
                slot = lax.rem(issued, RING)
                @pl.when(issued >= RING)
                def _():
                    pltpu.make_async_copy(
                        out_hbm.at[pl.ds(0, D)], ring_v.at[0], osem).wait()
                for g in range(D // 16):
                    val = plsc.load_gather(blk_ref, [g * 16 + lane, col_v])
                    ring_v[slot, pl.ds(g * 16, 16)] = val
                pltpu.async_copy(
                    ring_v.at[slot], out_hbm.at[pl.ds(pos_s * D, D)], osem)
                return issued + 1

            return lax.fori_loop(0, nh, hit_body, issued)

        return lax.fori_loop(0, ncv, cand_body, issued)

    def blk_body(j, issued):
        buf = lax.rem(j, 2)
        boff = lo + j * W
        pltpu.make_async_copy(
            ut_hbm.at[:, pl.ds(0, W)], ublk_v.at[buf], usem).wait()
        pltpu.make_async_copy(
            it_hbm.at[:, pl.ds(0, W)], iblk_v.at[buf], isem).wait()
        issued = scan_block(ublk_v.at[buf], j, boff, usidx_v, uspos_v,
                            ucntv, uout_hbm, issued)
        issued = scan_block(iblk_v.at[buf], j, boff, isidx_v, ispos_v,
                            icntv, iout_hbm, issued)
        @pl.when(j + 2 < nblk)
        def _():
            fire(j + 2, buf)
        return issued

    issued = lax.fori_loop(0, nblk, blk_body, jnp.int32(0))

    def tail_scan(idx_ref, tail_ref, out_hbm, issued):
        def body(v, issued):
            base = wid * BPW + v * 16
            idx = idx_ref[pl.ds(base, 16)]
            m = idx >= TAIL0
            nh = popcnt(m)
            plsc.store_compressed(hidx_v.at[pl.ds(0, 16)],
                                  (idx - TAIL0) * D, mask=m)
            plsc.store_compressed(hpos_v.at[pl.ds(0, 16)], base + lane, mask=m)

            def hit_body(h, issued):
                adr_s = hidx_v[pl.ds(h, 16)][0]
                pos_s = hpos_v[pl.ds(h, 16)][0]
                slot = lax.rem(issued, RING)
                @pl.when(issued >= RING)
                def _():
                    pltpu.make_async_copy(
                        out_hbm.at[pl.ds(0, D)], ring_v.at[0], osem).wait()
                for g in range(D // 16):
                    val = plsc.load_gather(tail_ref,
                                           [adr_s + g * 16 + lane])
                    ring_v[slot, pl.ds(g * 16, 16)] = val
                pltpu.async_copy(
                    ring_v.at[slot], out_hbm.at[pl.ds(pos_s * D, D)], osem)
                return issued + 1

            return lax.fori_loop(0, nh, hit_body, issued)
        return lax.fori_loop(0, BPW // 16, body, issued)

    issued = tail_scan(uidx_v, utail_v, uout_hbm, issued)
    issued = tail_scan(iidx_v, itail_v, iout_hbm, issued)

    def drain(_, carry):
        pltpu.make_async_copy(
            uout_hbm.at[pl.ds(0, D)], ring_v.at[0], osem).wait()
        return carry
    lax.fori_loop(0, jnp.minimum(issued, RING), drain, jnp.int32(0))


@functools.partial(
    pl.kernel,
    mesh=_mesh,
    compiler_params=pltpu.CompilerParams(needs_layout_passes=False),
    out_type=(
        jax.ShapeDtypeStruct((D, B), jnp.float32),
        jax.ShapeDtypeStruct((D, B), jnp.float32),
    ),
    scratch_types=[
        pltpu.VMEM((BPW * D,), jnp.float32),
        pltpu.VMEM((D, BPW), jnp.float32),
        pltpu.SemaphoreType.DMA,
    ],
)
def _retile(u1d_hbm, i1d_hbm, uout_hbm, iout_hbm, flat_v, outT_v, sem):
    wid = lax.axis_index("s") * 2 + lax.axis_index("c")
    lane = lax.iota(jnp.int32, 16)

    def one(src_hbm, dst_hbm):
        pltpu.sync_copy(src_hbm.at[pl.ds(wid * BPW * D, BPW * D)], flat_v)

        def body(g, carry):
            for c in range(D):
                val = plsc.load_gather(
                    flat_v, [(g * 16 + lane) * D + c])
                outT_v[c, pl.ds(g * 16, 16)] = val
            return carry
        lax.fori_loop(0, BPW // 16, body, 0)
        pltpu.sync_copy(outT_v, dst_hbm.at[:, pl.ds(wid * BPW, BPW)])

    one(u1d_hbm, uout_hbm)
    one(i1d_hbm, iout_hbm)


@jax.jit
def kernel(users, items, user_table, item_table):
    utail = user_table[TAIL0:].reshape(-1)
    itail = item_table[TAIL0:].reshape(-1)
    u1d, i1d = _stream_compact(users, items, user_table.T, item_table.T,
                               utail, itail)
    return (u1d.reshape(B, D), i1d.reshape(B, D))
